# baseline (speedup 1.0000x reference)
import functools

import jax
import jax.numpy as jnp
from jax import lax
from jax.experimental import pallas as pl
from jax.experimental.pallas import tpu as pltpu
from jax.experimental.pallas import tpu_sc as plsc

B, C, H, W = 2, 96, 224, 224
OC, K, WIN = 96, 3, 7
PAD = (WIN - 1) // 2
KK = K * K
NWIN = WIN * WIN
HW = H * W
HpL, WpL = 240, 256
CP = 128
NTOT = HW * KK
NC, NS = 2, 16
NW = NC * NS
NPW = NTOT // NW
CH = 112
NCH = NPW // CH
NBUF = 6
NGRP = NCH // NBUF
RB = 8
TP = 512


def _reflect_h(t):
    return jnp.where(t < 0, -t, jnp.where(t >= H, 2 * H - 2 - t, t))


def _reflect_w(t):
    return jnp.where(t < 0, -t, jnp.where(t >= W, 2 * W - 2 - t, t))


def _topk_body(lpa_ref, lpb_ref, idx_ref):
    b = pl.program_id(0)
    ib = pl.program_id(1)
    lw = jnp.concatenate([lpa_ref[0], lpb_ref[0]], axis=0)
    lu = jnp.stack(
        [lw[dy : dy + RB, dx : dx + W] for dy in range(WIN) for dx in range(WIN)],
        axis=0,
    )
    oid = lax.broadcasted_iota(jnp.int32, (NWIN, 1, 1), 0)
    rank = jnp.zeros((NWIN, RB, W), jnp.int32)
    for o2 in range(NWIN):
        lo = lu[o2][None]
        before = (lo < lu) | ((lo == lu) & (oid > o2))
        rank = rank + before.astype(jnp.int32)
    i0 = ib * RB
    ii = lax.broadcasted_iota(jnp.int32, (RB, 1), 0) + i0
    jj = lax.broadcasted_iota(jnp.int32, (1, W), 1)
    rows = [_reflect_h(ii + (dy - PAD)) * W for dy in range(WIN)]
    cols = [_reflect_w(jj + (dx - PAD)) for dx in range(WIN)]
    addrs = jnp.stack(
        [b * HW + rows[o // WIN] + cols[o % WIN] for o in range(NWIN)], axis=0
    )
    outs = []
    for r in range(KK):
        acc = jnp.zeros((RB, W), jnp.int32)
        for o in range(NWIN):
            acc = acc + jnp.where(rank[o] == r, addrs[o], 0)
        outs.append(acc)
    idx_ref[...] = jnp.stack(outs, axis=0).reshape(KK, 1, RB, W)


def _topk_call(lp):
    return pl.pallas_call(
        _topk_body,
        grid=(1, H // RB),
        in_specs=[
            pl.BlockSpec((1, RB, WpL), lambda b, i: (b, i, 0)),
            pl.BlockSpec((1, RB, WpL), lambda b, i: (b, i + 1, 0)),
        ],
        out_specs=pl.BlockSpec((KK, 1, RB, W), lambda b, i: (0, b, i, 0)),
        out_shape=jax.ShapeDtypeStruct((KK, 1, H, W), jnp.int32),
    )(lp, lp)


def _sc_gather(table, idx2):
    mesh = plsc.VectorSubcoreMesh(core_axis_name="c", subcore_axis_name="s")

    @functools.partial(
        pl.kernel,
        out_type=jax.ShapeDtypeStruct((NTOT, CP), jnp.float32),
        mesh=mesh,
        scratch_types=[
            pltpu.VMEM((NPW,), jnp.int32),
            *[pltpu.VMEM((CH, CP), jnp.float32) for _ in range(NBUF)],
            *[pltpu.SemaphoreType.DMA for _ in range(2 * NBUF)],
        ],
    )
    def run(table_hbm, idx_hbm, out_hbm, idx_v, *rest):
        bufs = rest[:NBUF]
        gsems = rest[NBUF : 2 * NBUF]
        ssems = rest[2 * NBUF : 3 * NBUF]
        wid = lax.axis_index("s") * NC + lax.axis_index("c")
        pltpu.sync_copy(idx_hbm.at[wid], idx_v)
        base = wid * NPW

        def start_gather(slot, j):
            pltpu.async_copy(
                table_hbm.at[idx_v.at[pl.ds(j * CH, CH)]], bufs[slot], gsems[slot]
            )

        def wait_gather(slot):
            pltpu.make_async_copy(
                table_hbm.at[pl.ds(0, CH)], bufs[slot], gsems[slot]
            ).wait()

        def start_store(slot, j):
            pltpu.async_copy(
                bufs[slot], out_hbm.at[pl.ds(base + j * CH, CH)], ssems[slot]
            )

        def wait_store(slot):
            pltpu.make_async_copy(
                bufs[slot], out_hbm.at[pl.ds(base, CH)], ssems[slot]
            ).wait()

        for slot in range(NBUF):
            start_gather(slot, slot)

        def group(g, carry):
            for slot in range(NBUF):
                wait_gather(slot)
                start_store(slot, g * NBUF + slot)
            for slot in range(NBUF):
                jn = (g + 1) * NBUF + slot

                @pl.when(jn < NCH)
                def _():
                    wait_store(slot)
                    start_gather(slot, jn)

            return carry

        lax.fori_loop(0, NGRP, group, 0)
        for slot in range(NBUF):
            wait_store(slot)

    return run(table, idx2)


def _mm_body(g_ref, w_ref, b_ref, o_ref):
    acc = b_ref[...].astype(jnp.float32)
    for r in range(KK):
        acc = acc + lax.dot_general(
            w_ref[r],
            g_ref[r],
            (((0,), (1,)), ((), ())),
            preferred_element_type=jnp.float32,
        )
    o_ref[...] = acc


def _mm_call(g3, w3, bias2):
    return pl.pallas_call(
        _mm_body,
        grid=(HW // TP,),
        in_specs=[
            pl.BlockSpec((KK, TP, CP), lambda i: (0, i, 0)),
            pl.BlockSpec((KK, CP, OC), lambda i: (0, 0, 0)),
            pl.BlockSpec((OC, 1), lambda i: (0, 0)),
        ],
        out_specs=pl.BlockSpec((OC, TP), lambda i: (0, i)),
        out_shape=jax.ShapeDtypeStruct((OC, HW), jnp.float32),
    )(g3, w3, bias2)


def kernel(x, l, weight, bias):
    lp = jnp.pad(
        l[:, 0],
        ((0, 0), (PAD, HpL - H - PAD), (PAD, WpL - W - PAD)),
        constant_values=999.0,
    )
    w3 = jnp.pad(
        jnp.transpose(weight.reshape(OC, C, KK), (2, 1, 0)), ((0, 0), (0, CP - C), (0, 0))
    )
    bias2 = bias.reshape(OC, 1)
    outs = []
    for b in range(B):
        table = jnp.pad(
            x[b].transpose(1, 2, 0), ((0, 0), (0, 0), (0, CP - C))
        ).reshape(HW, CP)
        idx = _topk_call(lp[b : b + 1])
        idx2 = idx.reshape(NW, NPW)
        g = _sc_gather(table, idx2)
        g3 = g.reshape(KK, HW, CP)
        outs.append(_mm_call(g3, w3, bias2))
    out = jnp.stack(outs)
    return out.reshape(B, OC, H, W)


# speedup vs baseline: 1.0079x; 1.0079x over previous
import functools

import jax
import jax.numpy as jnp
from jax import lax
from jax.experimental import pallas as pl
from jax.experimental.pallas import tpu as pltpu
from jax.experimental.pallas import tpu_sc as plsc

B, C, H, W = 2, 96, 224, 224
OC, K, WIN = 96, 3, 7
PAD = (WIN - 1) // 2
KK = K * K
NWIN = WIN * WIN
HW = H * W
HpL, WpL = 240, 256
CP = 128
NTOT = HW * KK
NC, NS = 2, 16
NW = NC * NS
NPW = NTOT // NW
CH = 112
NCH = NPW // CH
NBUF = 7
NGRP = NCH // NBUF
RB = 8
TP = 512


def _reflect_h(t):
    return jnp.where(t < 0, -t, jnp.where(t >= H, 2 * H - 2 - t, t))


def _reflect_w(t):
    return jnp.where(t < 0, -t, jnp.where(t >= W, 2 * W - 2 - t, t))


def _topk_body(lpa_ref, lpb_ref, idx_ref):
    b = pl.program_id(0)
    ib = pl.program_id(1)
    lw = jnp.concatenate([lpa_ref[0], lpb_ref[0]], axis=0)
    lu = jnp.stack(
        [lw[dy : dy + RB, dx : dx + W] for dy in range(WIN) for dx in range(WIN)],
        axis=0,
    )
    oid = lax.broadcasted_iota(jnp.int32, (NWIN, 1, 1), 0)
    rank = jnp.zeros((NWIN, RB, W), jnp.int32)
    for o2 in range(NWIN):
        lo = lu[o2][None]
        before = (lo < lu) | ((lo == lu) & (oid > o2))
        rank = rank + before.astype(jnp.int32)
    i0 = ib * RB
    ii = lax.broadcasted_iota(jnp.int32, (RB, 1), 0) + i0
    jj = lax.broadcasted_iota(jnp.int32, (1, W), 1)
    rows = [_reflect_h(ii + (dy - PAD)) * W for dy in range(WIN)]
    cols = [_reflect_w(jj + (dx - PAD)) for dx in range(WIN)]
    addrs = jnp.stack(
        [b * HW + rows[o // WIN] + cols[o % WIN] for o in range(NWIN)], axis=0
    )
    outs = []
    for r in range(KK):
        acc = jnp.zeros((RB, W), jnp.int32)
        for o in range(NWIN):
            acc = acc + jnp.where(rank[o] == r, addrs[o], 0)
        outs.append(acc)
    idx_ref[...] = jnp.stack(outs, axis=0).reshape(KK, 1, RB, W)


def _topk_call(lp):
    return pl.pallas_call(
        _topk_body,
        grid=(1, H // RB),
        in_specs=[
            pl.BlockSpec((1, RB, WpL), lambda b, i: (b, i, 0)),
            pl.BlockSpec((1, RB, WpL), lambda b, i: (b, i + 1, 0)),
        ],
        out_specs=pl.BlockSpec((KK, 1, RB, W), lambda b, i: (0, b, i, 0)),
        out_shape=jax.ShapeDtypeStruct((KK, 1, H, W), jnp.int32),
    )(lp, lp)


def _sc_gather(table, idx2):
    mesh = plsc.VectorSubcoreMesh(core_axis_name="c", subcore_axis_name="s")

    @functools.partial(
        pl.kernel,
        out_type=jax.ShapeDtypeStruct((NTOT, CP), jnp.float32),
        mesh=mesh,
        scratch_types=[
            pltpu.VMEM((NPW,), jnp.int32),
            *[pltpu.VMEM((CH, CP), jnp.float32) for _ in range(NBUF)],
            *[pltpu.SemaphoreType.DMA for _ in range(2 * NBUF)],
        ],
    )
    def run(table_hbm, idx_hbm, out_hbm, idx_v, *rest):
        bufs = rest[:NBUF]
        gsems = rest[NBUF : 2 * NBUF]
        ssems = rest[2 * NBUF : 3 * NBUF]
        wid = lax.axis_index("s") * NC + lax.axis_index("c")
        pltpu.sync_copy(idx_hbm.at[wid], idx_v)
        base = wid * NPW

        def start_gather(slot, j):
            pltpu.async_copy(
                table_hbm.at[idx_v.at[pl.ds(j * CH, CH)]], bufs[slot], gsems[slot]
            )

        def wait_gather(slot):
            pltpu.make_async_copy(
                table_hbm.at[pl.ds(0, CH)], bufs[slot], gsems[slot]
            ).wait()

        def start_store(slot, j):
            pltpu.async_copy(
                bufs[slot], out_hbm.at[pl.ds(base + j * CH, CH)], ssems[slot]
            )

        def wait_store(slot):
            pltpu.make_async_copy(
                bufs[slot], out_hbm.at[pl.ds(base, CH)], ssems[slot]
            ).wait()

        for slot in range(NBUF):
            start_gather(slot, slot)

        def group(g, carry):
            for slot in range(NBUF):
                wait_gather(slot)
                start_store(slot, g * NBUF + slot)
            for slot in range(NBUF):
                jn = (g + 1) * NBUF + slot

                @pl.when(jn < NCH)
                def _():
                    wait_store(slot)
                    start_gather(slot, jn)

            return carry

        lax.fori_loop(0, NGRP, group, 0)
        for slot in range(NBUF):
            wait_store(slot)

    return run(table, idx2)


def _mm_body(g_ref, w_ref, b_ref, o_ref):
    acc = b_ref[...].astype(jnp.float32)
    for r in range(KK):
        acc = acc + jnp.dot(
            g_ref[r], w_ref[r], preferred_element_type=jnp.float32
        )
    o_ref[...] = acc


def _mm_call(g3, w3, bias2):
    return pl.pallas_call(
        _mm_body,
        grid=(HW // TP,),
        in_specs=[
            pl.BlockSpec((KK, TP, CP), lambda i: (0, i, 0)),
            pl.BlockSpec((KK, CP, OC), lambda i: (0, 0, 0)),
            pl.BlockSpec((1, OC), lambda i: (0, 0)),
        ],
        out_specs=pl.BlockSpec((TP, OC), lambda i: (i, 0)),
        out_shape=jax.ShapeDtypeStruct((HW, OC), jnp.float32),
    )(g3, w3, bias2)


def kernel(x, l, weight, bias):
    lp = jnp.pad(
        l[:, 0],
        ((0, 0), (PAD, HpL - H - PAD), (PAD, WpL - W - PAD)),
        constant_values=999.0,
    )
    w3 = jnp.pad(
        jnp.transpose(weight.reshape(OC, C, KK), (2, 1, 0)), ((0, 0), (0, CP - C), (0, 0))
    )
    bias2 = bias.reshape(1, OC)
    outs = []
    for b in range(B):
        table = jnp.pad(
            x[b].transpose(1, 2, 0), ((0, 0), (0, 0), (0, CP - C))
        ).reshape(HW, CP)
        idx = _topk_call(lp[b : b + 1])
        idx2 = idx.reshape(NW, NPW)
        g = _sc_gather(table, idx2)
        g3 = g.reshape(KK, HW, CP)
        outs.append(_mm_call(g3, w3, bias2))
    out = jnp.stack(outs)
    return out.reshape(B, H, W, OC).transpose(0, 3, 1, 2)


# speedup vs baseline: 1.0110x; 1.0031x over previous
import functools

import jax
import jax.numpy as jnp
from jax import lax
from jax.experimental import pallas as pl
from jax.experimental.pallas import tpu as pltpu
from jax.experimental.pallas import tpu_sc as plsc

B, C, H, W = 2, 96, 224, 224
OC, K, WIN = 96, 3, 7
PAD = (WIN - 1) // 2
KK = K * K
NWIN = WIN * WIN
HW = H * W
HpL, WpL = 240, 256
CP = 128
NTOT = HW * KK
NC, NS = 2, 16
NW = NC * NS
NPW = NTOT // NW
CH = 112
NCH = NPW // CH
NBUF = 6
NGRP = NCH // NBUF
RB = 8
TP = 512


def _reflect_h(t):
    return jnp.where(t < 0, -t, jnp.where(t >= H, 2 * H - 2 - t, t))


def _reflect_w(t):
    return jnp.where(t < 0, -t, jnp.where(t >= W, 2 * W - 2 - t, t))


def _topk_body(lpa_ref, lpb_ref, idx_ref):
    b = pl.program_id(0)
    ib = pl.program_id(1)
    lw = jnp.concatenate([lpa_ref[0], lpb_ref[0]], axis=0)
    lu = jnp.stack(
        [lw[dy : dy + RB, dx : dx + W] for dy in range(WIN) for dx in range(WIN)],
        axis=0,
    )
    oid = lax.broadcasted_iota(jnp.int32, (NWIN, 1, 1), 0)
    rank = jnp.zeros((NWIN, RB, W), jnp.int32)
    for o2 in range(NWIN):
        lo = lu[o2][None]
        before = (lo < lu) | ((lo == lu) & (oid > o2))
        rank = rank + before.astype(jnp.int32)
    i0 = ib * RB
    ii = lax.broadcasted_iota(jnp.int32, (RB, 1), 0) + i0
    jj = lax.broadcasted_iota(jnp.int32, (1, W), 1)
    rows = [_reflect_h(ii + (dy - PAD)) * W for dy in range(WIN)]
    cols = [_reflect_w(jj + (dx - PAD)) for dx in range(WIN)]
    addrs = jnp.stack(
        [b * HW + rows[o // WIN] + cols[o % WIN] for o in range(NWIN)], axis=0
    )
    outs = []
    for r in range(KK):
        acc = jnp.zeros((RB, W), jnp.int32)
        for o in range(NWIN):
            acc = acc + jnp.where(rank[o] == r, addrs[o], 0)
        outs.append(acc)
    idx_ref[...] = jnp.stack(outs, axis=0).reshape(KK, 1, RB, W)


def _topk_call(lp):
    return pl.pallas_call(
        _topk_body,
        grid=(1, H // RB),
        in_specs=[
            pl.BlockSpec((1, RB, WpL), lambda b, i: (b, i, 0)),
            pl.BlockSpec((1, RB, WpL), lambda b, i: (b, i + 1, 0)),
        ],
        out_specs=pl.BlockSpec((KK, 1, RB, W), lambda b, i: (0, b, i, 0)),
        out_shape=jax.ShapeDtypeStruct((KK, 1, H, W), jnp.int32),
    )(lp, lp)


def _sc_gather(table, idx2):
    mesh = plsc.VectorSubcoreMesh(core_axis_name="c", subcore_axis_name="s")

    @functools.partial(
        pl.kernel,
        out_type=jax.ShapeDtypeStruct((NTOT, CP), jnp.float32),
        mesh=mesh,
        scratch_types=[
            pltpu.VMEM((NPW,), jnp.int32),
            *[pltpu.VMEM((CH, CP), jnp.float32) for _ in range(NBUF)],
            *[pltpu.SemaphoreType.DMA for _ in range(2 * NBUF)],
        ],
    )
    def run(table_hbm, idx_hbm, out_hbm, idx_v, *rest):
        bufs = rest[:NBUF]
        gsems = rest[NBUF : 2 * NBUF]
        ssems = rest[2 * NBUF : 3 * NBUF]
        wid = lax.axis_index("s") * NC + lax.axis_index("c")
        pltpu.sync_copy(idx_hbm.at[wid], idx_v)
        base = wid * NPW

        def start_gather(slot, j):
            pltpu.async_copy(
                table_hbm.at[idx_v.at[pl.ds(j * CH, CH)]], bufs[slot], gsems[slot]
            )

        def wait_gather(slot):
            pltpu.make_async_copy(
                table_hbm.at[pl.ds(0, CH)], bufs[slot], gsems[slot]
            ).wait()

        def start_store(slot, j):
            pltpu.async_copy(
                bufs[slot], out_hbm.at[pl.ds(base + j * CH, CH)], ssems[slot]
            )

        def wait_store(slot):
            pltpu.make_async_copy(
                bufs[slot], out_hbm.at[pl.ds(base, CH)], ssems[slot]
            ).wait()

        for slot in range(NBUF):
            start_gather(slot, slot)

        def group(g, carry):
            for slot in range(NBUF):
                wait_gather(slot)
                start_store(slot, g * NBUF + slot)
            for slot in range(NBUF):
                jn = (g + 1) * NBUF + slot

                @pl.when(jn < NCH)
                def _():
                    wait_store(slot)
                    start_gather(slot, jn)

            return carry

        lax.fori_loop(0, NGRP, group, 0)
        for slot in range(NBUF):
            wait_store(slot)

    return run(table, idx2)


def _mm_body(g_ref, w_ref, b_ref, o_ref):
    acc = b_ref[...].astype(jnp.float32)
    for r in range(KK):
        acc = acc + jnp.dot(
            g_ref[r], w_ref[r], preferred_element_type=jnp.float32
        )
    o_ref[...] = acc


def _mm_call(g3, w3, bias2):
    return pl.pallas_call(
        _mm_body,
        grid=(HW // TP,),
        in_specs=[
            pl.BlockSpec((KK, TP, CP), lambda i: (0, i, 0)),
            pl.BlockSpec((KK, CP, OC), lambda i: (0, 0, 0)),
            pl.BlockSpec((1, OC), lambda i: (0, 0)),
        ],
        out_specs=pl.BlockSpec((TP, OC), lambda i: (i, 0)),
        out_shape=jax.ShapeDtypeStruct((HW, OC), jnp.float32),
    )(g3, w3, bias2)


def kernel(x, l, weight, bias):
    lp = jnp.pad(
        l[:, 0],
        ((0, 0), (PAD, HpL - H - PAD), (PAD, WpL - W - PAD)),
        constant_values=999.0,
    )
    w3 = jnp.pad(
        jnp.transpose(weight.reshape(OC, C, KK), (2, 1, 0)), ((0, 0), (0, CP - C), (0, 0))
    )
    bias2 = bias.reshape(1, OC)
    outs = []
    for b in range(B):
        table = jnp.pad(
            x[b].transpose(1, 2, 0), ((0, 0), (0, 0), (0, CP - C))
        ).reshape(HW, CP)
        idx = _topk_call(lp[b : b + 1])
        idx2 = idx.reshape(NW, NPW)
        g = _sc_gather(table, idx2)
        g3 = g.reshape(KK, HW, CP)
        outs.append(_mm_call(g3, w3, bias2))
    out = jnp.stack(outs)
    return out.reshape(B, H, W, OC).transpose(0, 3, 1, 2)


# speedup vs baseline: 1.0334x; 1.0222x over previous
import functools

import jax
import jax.numpy as jnp
from jax import lax
from jax.experimental import pallas as pl
from jax.experimental.pallas import tpu as pltpu
from jax.experimental.pallas import tpu_sc as plsc

B, C, H, W = 2, 96, 224, 224
OC, K, WIN = 96, 3, 7
PAD = (WIN - 1) // 2
KK = K * K
NWIN = WIN * WIN
HW = H * W
HpL, WpL = 240, 256
CP = 128
NTOT = HW * KK
NC, NS = 2, 16
NW = NC * NS
NPW = NTOT // NW
CH = 112
NCH = NPW // CH
NBUF = 6
NGRP = NCH // NBUF
RB = 8
TP = 1024


def _reflect_h(t):
    return jnp.where(t < 0, -t, jnp.where(t >= H, 2 * H - 2 - t, t))


def _reflect_w(t):
    return jnp.where(t < 0, -t, jnp.where(t >= W, 2 * W - 2 - t, t))


def _topk_body(lpa_ref, lpb_ref, idx_ref):
    b = pl.program_id(0)
    ib = pl.program_id(1)
    lw = jnp.concatenate([lpa_ref[0], lpb_ref[0]], axis=0)
    lu = jnp.stack(
        [lw[dy : dy + RB, dx : dx + W] for dy in range(WIN) for dx in range(WIN)],
        axis=0,
    )
    oid = lax.broadcasted_iota(jnp.int32, (NWIN, 1, 1), 0)
    rank = jnp.zeros((NWIN, RB, W), jnp.int32)
    for o2 in range(NWIN):
        lo = lu[o2][None]
        before = (lo < lu) | ((lo == lu) & (oid > o2))
        rank = rank + before.astype(jnp.int32)
    i0 = ib * RB
    ii = lax.broadcasted_iota(jnp.int32, (RB, 1), 0) + i0
    jj = lax.broadcasted_iota(jnp.int32, (1, W), 1)
    rows = [_reflect_h(ii + (dy - PAD)) * W for dy in range(WIN)]
    cols = [_reflect_w(jj + (dx - PAD)) for dx in range(WIN)]
    addrs = jnp.stack(
        [b * HW + rows[o // WIN] + cols[o % WIN] for o in range(NWIN)], axis=0
    )
    outs = []
    for r in range(KK):
        acc = jnp.zeros((RB, W), jnp.int32)
        for o in range(NWIN):
            acc = acc + jnp.where(rank[o] == r, addrs[o], 0)
        outs.append(acc)
    idx_ref[...] = jnp.stack(outs, axis=0).reshape(KK, 1, RB, W)


def _topk_call(lp):
    return pl.pallas_call(
        _topk_body,
        grid=(1, H // RB),
        in_specs=[
            pl.BlockSpec((1, RB, WpL), lambda b, i: (b, i, 0)),
            pl.BlockSpec((1, RB, WpL), lambda b, i: (b, i + 1, 0)),
        ],
        out_specs=pl.BlockSpec((KK, 1, RB, W), lambda b, i: (0, b, i, 0)),
        out_shape=jax.ShapeDtypeStruct((KK, 1, H, W), jnp.int32),
    )(lp, lp)


def _sc_gather(table, idx2):
    mesh = plsc.VectorSubcoreMesh(core_axis_name="c", subcore_axis_name="s")

    @functools.partial(
        pl.kernel,
        out_type=jax.ShapeDtypeStruct((NTOT, CP), jnp.float32),
        mesh=mesh,
        scratch_types=[
            pltpu.VMEM((NPW,), jnp.int32),
            *[pltpu.VMEM((CH, CP), jnp.float32) for _ in range(NBUF)],
            *[pltpu.SemaphoreType.DMA for _ in range(2 * NBUF)],
        ],
    )
    def run(table_hbm, idx_hbm, out_hbm, idx_v, *rest):
        bufs = rest[:NBUF]
        gsems = rest[NBUF : 2 * NBUF]
        ssems = rest[2 * NBUF : 3 * NBUF]
        wid = lax.axis_index("s") * NC + lax.axis_index("c")
        pltpu.sync_copy(idx_hbm.at[wid], idx_v)
        base = wid * NPW

        def start_gather(slot, j):
            pltpu.async_copy(
                table_hbm.at[idx_v.at[pl.ds(j * CH, CH)]], bufs[slot], gsems[slot]
            )

        def wait_gather(slot):
            pltpu.make_async_copy(
                table_hbm.at[pl.ds(0, CH)], bufs[slot], gsems[slot]
            ).wait()

        def start_store(slot, j):
            pltpu.async_copy(
                bufs[slot], out_hbm.at[pl.ds(base + j * CH, CH)], ssems[slot]
            )

        def wait_store(slot):
            pltpu.make_async_copy(
                bufs[slot], out_hbm.at[pl.ds(base, CH)], ssems[slot]
            ).wait()

        for slot in range(NBUF):
            start_gather(slot, slot)

        def group(g, carry):
            for slot in range(NBUF):
                wait_gather(slot)
                start_store(slot, g * NBUF + slot)
            for slot in range(NBUF):
                jn = (g + 1) * NBUF + slot

                @pl.when(jn < NCH)
                def _():
                    wait_store(slot)
                    start_gather(slot, jn)

            return carry

        lax.fori_loop(0, NGRP, group, 0)
        for slot in range(NBUF):
            wait_store(slot)

    return run(table, idx2)


def _mm_body(g_ref, w_ref, b_ref, o_ref):
    acc = b_ref[...].astype(jnp.float32)
    for r in range(KK):
        acc = acc + jnp.dot(
            g_ref[r], w_ref[r], preferred_element_type=jnp.float32
        )
    o_ref[...] = acc


def _mm_call(g3, w3, bias2):
    return pl.pallas_call(
        _mm_body,
        grid=(HW // TP,),
        in_specs=[
            pl.BlockSpec((KK, TP, CP), lambda i: (0, i, 0)),
            pl.BlockSpec((KK, CP, OC), lambda i: (0, 0, 0)),
            pl.BlockSpec((1, OC), lambda i: (0, 0)),
        ],
        out_specs=pl.BlockSpec((TP, OC), lambda i: (i, 0)),
        out_shape=jax.ShapeDtypeStruct((HW, OC), jnp.float32),
    )(g3, w3, bias2)


def kernel(x, l, weight, bias):
    lp = jnp.pad(
        l[:, 0],
        ((0, 0), (PAD, HpL - H - PAD), (PAD, WpL - W - PAD)),
        constant_values=999.0,
    )
    w3 = jnp.pad(
        jnp.transpose(weight.reshape(OC, C, KK), (2, 1, 0)), ((0, 0), (0, CP - C), (0, 0))
    )
    bias2 = bias.reshape(1, OC)
    outs = []
    for b in range(B):
        table = jnp.pad(
            x[b].transpose(1, 2, 0), ((0, 0), (0, 0), (0, CP - C))
        ).reshape(HW, CP)
        idx = _topk_call(lp[b : b + 1])
        idx2 = idx.reshape(NW, NPW)
        g = _sc_gather(table, idx2)
        g3 = g.reshape(KK, HW, CP)
        outs.append(_mm_call(g3, w3, bias2))
    out = jnp.stack(outs)
    return out.reshape(B, H, W, OC).transpose(0, 3, 1, 2)


# speedup vs baseline: 1.0456x; 1.0118x over previous
import functools

import jax
import jax.numpy as jnp
from jax import lax
from jax.experimental import pallas as pl
from jax.experimental.pallas import tpu as pltpu
from jax.experimental.pallas import tpu_sc as plsc

B, C, H, W = 2, 96, 224, 224
OC, K, WIN = 96, 3, 7
PAD = (WIN - 1) // 2
KK = K * K
NWIN = WIN * WIN
HW = H * W
HpL, WpL = 240, 256
CP = 128
NTOT = HW * KK
NC, NS = 2, 16
NW = NC * NS
NPW = NTOT // NW
CH = 112
NCH = NPW // CH
NBUF = 6
NGRP = NCH // NBUF
RB = 8
TP = 1792


def _reflect_h(t):
    return jnp.where(t < 0, -t, jnp.where(t >= H, 2 * H - 2 - t, t))


def _reflect_w(t):
    return jnp.where(t < 0, -t, jnp.where(t >= W, 2 * W - 2 - t, t))


def _topk_body(lpa_ref, lpb_ref, idx_ref):
    b = pl.program_id(0)
    ib = pl.program_id(1)
    lw = jnp.concatenate([lpa_ref[0], lpb_ref[0]], axis=0)
    lu = jnp.stack(
        [lw[dy : dy + RB, dx : dx + W] for dy in range(WIN) for dx in range(WIN)],
        axis=0,
    )
    oid = lax.broadcasted_iota(jnp.int32, (NWIN, 1, 1), 0)
    rank = jnp.zeros((NWIN, RB, W), jnp.int32)
    for o2 in range(NWIN):
        lo = lu[o2][None]
        before = (lo < lu) | ((lo == lu) & (oid > o2))
        rank = rank + before.astype(jnp.int32)
    i0 = ib * RB
    ii = lax.broadcasted_iota(jnp.int32, (RB, 1), 0) + i0
    jj = lax.broadcasted_iota(jnp.int32, (1, W), 1)
    rows = [_reflect_h(ii + (dy - PAD)) * W for dy in range(WIN)]
    cols = [_reflect_w(jj + (dx - PAD)) for dx in range(WIN)]
    addrs = jnp.stack(
        [b * HW + rows[o // WIN] + cols[o % WIN] for o in range(NWIN)], axis=0
    )
    outs = []
    for r in range(KK):
        acc = jnp.zeros((RB, W), jnp.int32)
        for o in range(NWIN):
            acc = acc + jnp.where(rank[o] == r, addrs[o], 0)
        outs.append(acc)
    idx_ref[...] = jnp.stack(outs, axis=0).reshape(KK, 1, RB, W)


def _topk_call(lp):
    return pl.pallas_call(
        _topk_body,
        grid=(1, H // RB),
        in_specs=[
            pl.BlockSpec((1, RB, WpL), lambda b, i: (b, i, 0)),
            pl.BlockSpec((1, RB, WpL), lambda b, i: (b, i + 1, 0)),
        ],
        out_specs=pl.BlockSpec((KK, 1, RB, W), lambda b, i: (0, b, i, 0)),
        out_shape=jax.ShapeDtypeStruct((KK, 1, H, W), jnp.int32),
    )(lp, lp)


def _sc_gather(table, idx2):
    mesh = plsc.VectorSubcoreMesh(core_axis_name="c", subcore_axis_name="s")

    @functools.partial(
        pl.kernel,
        out_type=jax.ShapeDtypeStruct((NTOT, CP), jnp.float32),
        mesh=mesh,
        scratch_types=[
            pltpu.VMEM((NPW,), jnp.int32),
            *[pltpu.VMEM((CH, CP), jnp.float32) for _ in range(NBUF)],
            *[pltpu.SemaphoreType.DMA for _ in range(2 * NBUF)],
        ],
    )
    def run(table_hbm, idx_hbm, out_hbm, idx_v, *rest):
        bufs = rest[:NBUF]
        gsems = rest[NBUF : 2 * NBUF]
        ssems = rest[2 * NBUF : 3 * NBUF]
        wid = lax.axis_index("s") * NC + lax.axis_index("c")
        pltpu.sync_copy(idx_hbm.at[wid], idx_v)
        base = wid * NPW

        def start_gather(slot, j):
            pltpu.async_copy(
                table_hbm.at[idx_v.at[pl.ds(j * CH, CH)]], bufs[slot], gsems[slot]
            )

        def wait_gather(slot):
            pltpu.make_async_copy(
                table_hbm.at[pl.ds(0, CH)], bufs[slot], gsems[slot]
            ).wait()

        def start_store(slot, j):
            pltpu.async_copy(
                bufs[slot], out_hbm.at[pl.ds(base + j * CH, CH)], ssems[slot]
            )

        def wait_store(slot):
            pltpu.make_async_copy(
                bufs[slot], out_hbm.at[pl.ds(base, CH)], ssems[slot]
            ).wait()

        for slot in range(NBUF):
            start_gather(slot, slot)

        def group(g, carry):
            for slot in range(NBUF):
                wait_gather(slot)
                start_store(slot, g * NBUF + slot)
            for slot in range(NBUF):
                jn = (g + 1) * NBUF + slot

                @pl.when(jn < NCH)
                def _():
                    wait_store(slot)
                    start_gather(slot, jn)

            return carry

        lax.fori_loop(0, NGRP, group, 0)
        for slot in range(NBUF):
            wait_store(slot)

    return run(table, idx2)


def _mm_body(g_ref, w_ref, b_ref, o_ref):
    acc = b_ref[...].astype(jnp.float32)
    for r in range(KK):
        acc = acc + jnp.dot(
            g_ref[r], w_ref[r], preferred_element_type=jnp.float32
        )
    o_ref[...] = acc


def _mm_call(g3, w3, bias2):
    return pl.pallas_call(
        _mm_body,
        grid=(HW // TP,),
        in_specs=[
            pl.BlockSpec((KK, TP, CP), lambda i: (0, i, 0)),
            pl.BlockSpec((KK, CP, OC), lambda i: (0, 0, 0)),
            pl.BlockSpec((1, OC), lambda i: (0, 0)),
        ],
        out_specs=pl.BlockSpec((TP, OC), lambda i: (i, 0)),
        out_shape=jax.ShapeDtypeStruct((HW, OC), jnp.float32),
    )(g3, w3, bias2)


def kernel(x, l, weight, bias):
    lp = jnp.pad(
        l[:, 0],
        ((0, 0), (PAD, HpL - H - PAD), (PAD, WpL - W - PAD)),
        constant_values=999.0,
    )
    w3 = jnp.pad(
        jnp.transpose(weight.reshape(OC, C, KK), (2, 1, 0)), ((0, 0), (0, CP - C), (0, 0))
    )
    bias2 = bias.reshape(1, OC)
    outs = []
    for b in range(B):
        table = jnp.pad(
            x[b].transpose(1, 2, 0), ((0, 0), (0, 0), (0, CP - C))
        ).reshape(HW, CP)
        idx = _topk_call(lp[b : b + 1])
        idx2 = idx.reshape(NW, NPW)
        g = _sc_gather(table, idx2)
        g3 = g.reshape(KK, HW, CP)
        outs.append(_mm_call(g3, w3, bias2))
    out = jnp.stack(outs)
    return out.reshape(B, H, W, OC).transpose(0, 3, 1, 2)
